# Initial kernel scaffold; baseline (speedup 1.0000x reference)
import jax
import jax.numpy as jnp
from jax.experimental import pallas as pl


def kernel(emb, Wt, Ut, Ws, Us, Wx, Ux, W_pred, b_pred, W_nt, codebook, x):
    raise NotImplementedError("write your pallas kernel here")


# speedup vs baseline: 4.6165x; 4.6165x over previous
import jax
import jax.numpy as jnp
from jax.experimental import pallas as pl
from jax.experimental.pallas import tpu as pltpu

_B, _S, _V, _K, _D, _H, _E = 64, 32, 8192, 1024, 256, 512, 256
_DEPTH = 2 * _S + 2
_PAD_ID = 0


def _recur_kernel(xe_ref, xm_ref, Wt_ref, Ut_ref, Ws_ref, Us_ref, Wnt_ref,
                  cb0_ref, cb1_ref, cb2_ref, cbT_ref, cbsq_ref,
                  hall_ref, pushes_ref, codes_ref):
    iota_d = jax.lax.broadcasted_iota(jnp.int32, (_B, _DEPTH), 1)
    iota_k = jax.lax.broadcasted_iota(jnp.int32, (_B, _K), 1)
    iota_s = jax.lax.broadcasted_iota(jnp.int32, (_B, _S), 1)
    xm = xm_ref[...]
    Wt = Wt_ref[...]
    Ut = Ut_ref[...]
    Ws = Ws_ref[...]
    Us = Us_ref[...]
    Wnt = Wnt_ref[...]
    cb0 = cb0_ref[...]
    cb1 = cb1_ref[...]
    cb2 = cb2_ref[...]
    cbT = cbT_ref[...]
    cbsq = cbsq_ref[...]

    def gather_rows(idx):
        onehot = (iota_k == idx).astype(jnp.float32)
        g = jnp.dot(onehot, cb0, preferred_element_type=jnp.float32)
        g = g + jnp.dot(onehot, cb1, preferred_element_type=jnp.float32)
        g = g + jnp.dot(onehot, cb2, preferred_element_type=jnp.float32)
        return g

    def quantize(c):
        csq = jnp.sum(c * c, axis=1, keepdims=True)
        dots = jnp.dot(c, cbT, preferred_element_type=jnp.float32,
                       precision=jax.lax.Precision.HIGHEST)
        d2 = (csq + cbsq) - 2.0 * dots
        m1 = jnp.min(d2, axis=1, keepdims=True)
        i1 = jnp.min(jnp.where(d2 == m1, iota_k, _K), axis=1, keepdims=True)
        d2b = jnp.where(iota_k == i1, jnp.float32(jnp.inf), d2)
        m2 = jnp.min(d2b, axis=1, keepdims=True)
        i2 = jnp.min(jnp.where(d2b == m2, iota_k, _K), axis=1, keepdims=True)
        e1 = gather_rows(i1)
        e2 = gather_rows(i2)
        f1 = c - e1
        f2 = c - e2
        dn1 = jnp.sqrt(jnp.sum(f1 * f1, axis=1, keepdims=True))
        dn2 = jnp.sqrt(jnp.sum(f2 * f2, axis=1, keepdims=True))
        better2 = (dn2 < dn1) | ((dn2 == dn1) & (i2 < i1))
        return jnp.where(better2, i2, i1)

    def push(stk, ptr, idx_val, mask):
        do = mask > 0
        sel = (iota_d == ptr) & do
        stk = jnp.where(sel, idx_val, stk)
        ptr = ptr + do.astype(jnp.int32)
        return stk, ptr

    def body(step, carry):
        stk, ptr, h_tok, h_stk = carry
        xe_s = xe_ref[pl.ds(step, 1)].reshape(_B, _E)
        msk = jnp.sum(jnp.where(iota_s == step, xm, 0), axis=1, keepdims=True)

        top_mask = (ptr > 0).astype(jnp.int32)
        idxm1 = jnp.maximum(ptr - 1, 0)
        sel = iota_d == idxm1
        top_idx = jnp.sum(jnp.where(sel, stk, 0), axis=1, keepdims=True)
        ptr = idxm1
        step_mask = msk * top_mask

        onehot = ((iota_k == top_idx) & (top_idx > 0)).astype(jnp.float32)
        top = jnp.dot(onehot, cb0, preferred_element_type=jnp.float32)
        top = top + jnp.dot(onehot, cb1, preferred_element_type=jnp.float32)
        top = top + jnp.dot(onehot, cb2, preferred_element_type=jnp.float32)

        inp = jnp.concatenate([xe_s, top], axis=1)
        h_tok = jnp.dot(inp, Wt) + jnp.dot(h_tok, Ut)
        h_stk = jnp.dot(inp, Ws) + jnp.dot(h_stk, Us)
        h_t = jnp.tanh(h_tok)
        h_nt = jnp.tanh(h_stk)
        cc = jnp.concatenate([h_nt, h_t], axis=1)
        codes = jnp.tanh(jnp.dot(cc, Wnt))

        qi0 = quantize(codes[:, :_D])
        qi1 = quantize(codes[:, _D:])
        q0m = qi0 * step_mask
        q1m = qi1 * step_mask
        stk, ptr = push(stk, ptr, qi0, q0m)
        stk, ptr = push(stk, ptr, qi1, q1m)

        hall_ref[pl.ds(step, 1)] = h_t[None]
        codes_ref[pl.ds(step, 1)] = codes[None]
        pq = jnp.concatenate([q0m[:, :, None], q1m[:, :, None]], axis=2)
        pushes_ref[:, pl.ds(step, 1), :] = pq
        return stk, ptr, h_tok, h_stk

    stk0 = jnp.zeros((_B, _DEPTH), jnp.int32)
    ptr0 = jnp.ones((_B, 1), jnp.int32)
    h0 = jnp.zeros((_B, _H), jnp.float32)
    jax.lax.fori_loop(0, _S, body, (stk0, ptr0, h0, h0))


def _logits_kernel(h_ref, w_ref, b_ref, o_ref):
    h = h_ref[0]
    acc = jnp.dot(h, w_ref[...]) + b_ref[...]
    o_ref[...] = acc.reshape(_B, 1, 1, -1)


def kernel(emb, Wt, Ut, Ws, Us, Wx, Ux, W_pred, b_pred, W_nt, codebook, x):
    del Wx, Ux
    xe = jnp.take(emb, x.T, axis=0)
    xm = (x != _PAD_ID).astype(jnp.int32)
    cbT = codebook.T
    cbsq = jnp.sum(codebook * codebook, axis=1)[None, :]
    msk = jnp.int32(-65536)
    cb0 = jax.lax.bitcast_convert_type(
        jax.lax.bitcast_convert_type(codebook, jnp.int32) & msk, jnp.float32)
    r1 = codebook - cb0
    cb1 = jax.lax.bitcast_convert_type(
        jax.lax.bitcast_convert_type(r1, jnp.int32) & msk, jnp.float32)
    cb2 = r1 - cb1

    hall, pushes, codes_sb = pl.pallas_call(
        _recur_kernel,
        out_shape=(
            jax.ShapeDtypeStruct((_S, _B, _H), jnp.float32),
            jax.ShapeDtypeStruct((_B, _S, 2), jnp.int32),
            jax.ShapeDtypeStruct((_S, _B, 2 * _D), jnp.float32),
        ),
    )(xe, xm, Wt, Ut, Ws, Us, W_nt, cb0, cb1, cb2, cbT, cbsq)

    TV = 512
    logits = pl.pallas_call(
        _logits_kernel,
        grid=(_S, _V // TV),
        in_specs=[
            pl.BlockSpec((1, _B, _H), lambda s, v: (s, 0, 0)),
            pl.BlockSpec((_H, TV), lambda s, v: (0, v)),
            pl.BlockSpec((1, TV), lambda s, v: (0, v)),
        ],
        out_specs=pl.BlockSpec((_B, 1, 1, TV), lambda s, v: (0, s, 0, v)),
        out_shape=jax.ShapeDtypeStruct((_B, _S, 1, _V), jnp.float32),
    )(hall, W_pred, b_pred[None, :])

    logits_all = logits.reshape(_B, _S, _V)
    raw_codes = jnp.transpose(codes_sb, (1, 0, 2)).reshape(_B, _S, 2, _D)
    return logits_all, pushes, raw_codes
